# baseline (speedup 1.0000x reference)
import functools

import jax
import jax.numpy as jnp
from jax import lax
from jax.experimental import pallas as pl
from jax.experimental.pallas import tpu as pltpu
from jax.experimental.pallas import tpu_sc as plsc

N_NODES = 10000
FEATS = 128
K = 25
E_RAW = 75
E_PAD = 80
W_PAD = 96

NW = 32
C = 4
N_PAD = 10240
CHUNKS = N_PAD // C
CHUNKS_PER_W = CHUNKS // NW


def _mm_body(h_ref, wt_ref, b_ref, o_ref):
    o_ref[...] = jnp.dot(h_ref[...], wt_ref[...],
                         preferred_element_type=jnp.float32) + b_ref[...]


def _tc_matmul(h, wt, b2):
    blk = 1024
    nb = h.shape[0] // blk
    kk = h.shape[1]
    return pl.pallas_call(
        _mm_body,
        grid=(nb,),
        in_specs=[
            pl.BlockSpec((blk, kk), lambda i: (i, 0)),
            pl.BlockSpec((kk, FEATS), lambda i: (0, 0)),
            pl.BlockSpec((1, FEATS), lambda i: (0, 0)),
        ],
        out_specs=pl.BlockSpec((blk, FEATS), lambda i: (i, 0)),
        out_shape=jax.ShapeDtypeStruct((h.shape[0], FEATS), jnp.float32),
    )(h, wt, b2)


def _sc_body(x_hbm, gidx_hbm, w_hbm, h_hbm,
             idx_v, w_v, rows_v, h_v, sem):
    wid = lax.axis_index("s") * 2 + lax.axis_index("c")

    lane = jnp.arange(16, dtype=jnp.int32)
    masks = [jnp.where(lane % 3 == t, 1.0, 0.0).astype(jnp.float32)
             for t in range(3)]

    def chunk_body(t, _):
        chunk = wid * CHUNKS_PER_W + t
        pltpu.sync_copy(gidx_hbm.at[chunk], idx_v)
        pltpu.sync_copy(w_hbm.at[chunk], w_v)
        copies = [
            pltpu.async_copy(x_hbm.at[idx_v.at[j]], rows_v.at[j], sem)
            for j in range(C)
        ]
        for cp in copies:
            cp.wait()
        for i in range(C):
            def k_body(kk, _2):
                wv = w_v[i, pl.ds(kk * 3, 16)]
                w3 = [jnp.full((16,), wv[t], jnp.float32) for t in range(3)]
                V = [w3[p % 3] * masks[0]
                     + w3[(p + 1) % 3] * masks[1]
                     + w3[(p + 2) % 3] * masks[2]
                     for p in range(3)]
                for f in range(8):
                    acc = V[f % 3] * rows_v[i, kk * 3, pl.ds(f * 16, 16)]
                    acc = acc + V[(2 + f) % 3] * rows_v[i, kk * 3 + 1,
                                                        pl.ds(f * 16, 16)]
                    acc = acc + V[(4 + f) % 3] * rows_v[i, kk * 3 + 2,
                                                        pl.ds(f * 16, 16)]
                    h_v[i, kk, pl.ds(f * 16, 16)] = acc
                return _2
            lax.fori_loop(0, K, k_body, None)
        pltpu.sync_copy(h_v, h_hbm.at[pl.ds(chunk * C, C)])
        return _

    lax.fori_loop(0, CHUNKS_PER_W, chunk_body, None)


_sc_gather = functools.partial(
    pl.kernel,
    out_type=jax.ShapeDtypeStruct((N_PAD, K, FEATS), jnp.float32),
    mesh=plsc.VectorSubcoreMesh(core_axis_name="c", subcore_axis_name="s"),
    scratch_types=[
        pltpu.VMEM((C, E_PAD), jnp.int32),
        pltpu.VMEM((C, W_PAD), jnp.float32),
        pltpu.VMEM((C, E_PAD, FEATS), jnp.float32),
        pltpu.VMEM((C, K, FEATS), jnp.float32),
        pltpu.SemaphoreType.DMA,
    ],
)(_sc_body)


def kernel(x, neigh_indices, neigh_weights, W, b):
    n = x.shape[0]
    gidx = neigh_indices - 1
    gidx = jnp.pad(gidx, ((0, N_PAD - n), (0, E_PAD - E_RAW)))
    wflat = neigh_weights.reshape(n, E_RAW)
    wflat = jnp.pad(wflat, ((0, N_PAD - n), (0, W_PAD - E_RAW)))

    h = _sc_gather(x,
                   gidx.reshape(CHUNKS, C, E_PAD),
                   wflat.reshape(CHUNKS, C, W_PAD))

    out = _tc_matmul(h.reshape(N_PAD, K * FEATS), W.T, b.reshape(1, FEATS))
    return out[:n]


# speedup vs baseline: 1.0796x; 1.0796x over previous
import jax
import jax.numpy as jnp
from jax import lax
from jax.experimental import pallas as pl
from jax.experimental.pallas import tpu as pltpu
from jax.experimental.pallas import tpu_sc as plsc

N_NODES = 10000
FEATS = 128
K = 25
E_RAW = 75
E_PAD = 80
W_PAD = 96

NW = 32
C = 4
N_PAD = 10240
CHUNKS = N_PAD // C
CHUNKS_PER_W = CHUNKS // NW


def _mm_body(h_ref, wt_ref, b_ref, o_ref):
    o_ref[...] = jnp.dot(h_ref[...], wt_ref[...],
                         preferred_element_type=jnp.float32) + b_ref[...]


def _tc_matmul(h, wt, b2):
    blk = 1024
    nb = h.shape[0] // blk
    kk = h.shape[1]
    return pl.pallas_call(
        _mm_body,
        grid=(nb,),
        in_specs=[
            pl.BlockSpec((blk, kk), lambda i: (i, 0)),
            pl.BlockSpec((kk, FEATS), lambda i: (0, 0)),
            pl.BlockSpec((1, FEATS), lambda i: (0, 0)),
        ],
        out_specs=pl.BlockSpec((blk, FEATS), lambda i: (i, 0)),
        out_shape=jax.ShapeDtypeStruct((h.shape[0], FEATS), jnp.float32),
    )(h, wt, b2)


def _sc_body(x_hbm, gidx_hbm, w_hbm, h_hbm,
             idx0, idx1, w0, w1, rows0, rows1, h_v, sem0, sem1):
    wid = lax.axis_index("s") * 2 + lax.axis_index("c")
    base = wid * CHUNKS_PER_W
    idx_v = (idx0, idx1)
    w_v = (w0, w1)
    rows_v = (rows0, rows1)
    sem = (sem0, sem1)

    lane = jnp.arange(16, dtype=jnp.int32)
    rotidx = [((lane + p) % 3)[:, None] for p in range(3)]
    gdn = lax.GatherDimensionNumbers(
        offset_dims=(), collapsed_slice_dims=(0,), start_index_map=(0,))

    def lane_gather(vec, idx):
        return lax.gather(vec, idx, dimension_numbers=gdn, slice_sizes=(1,),
                          mode=lax.GatherScatterMode.PROMISE_IN_BOUNDS)

    def prefetch(t, buf):
        chunk = jnp.minimum(base + t, CHUNKS - 1)
        pltpu.sync_copy(gidx_hbm.at[chunk], idx_v[buf])
        pltpu.sync_copy(w_hbm.at[chunk], w_v[buf])
        for j in range(C):
            pltpu.async_copy(x_hbm.at[idx_v[buf].at[j]],
                             rows_v[buf].at[j], sem[buf])

    def wait_rows(buf):
        for j in range(C):
            pltpu.make_async_copy(x_hbm.at[idx_v[buf].at[j]],
                                  rows_v[buf].at[j], sem[buf]).wait()

    def compute(t, buf):
        for i in range(C):
            def k_body(kk):
                wv = w_v[buf][i, pl.ds(kk * 3, 16)]
                V = [lane_gather(wv, rotidx[p]) for p in range(3)]
                accs = []
                for f in range(8):
                    r0 = rows_v[buf][i, kk * 3, pl.ds(f * 16, 16)]
                    r1 = rows_v[buf][i, kk * 3 + 1, pl.ds(f * 16, 16)]
                    r2 = rows_v[buf][i, kk * 3 + 2, pl.ds(f * 16, 16)]
                    accs.append(V[f % 3] * r0
                                + V[(f + 2) % 3] * r1
                                + V[(f + 1) % 3] * r2)
                for f in range(8):
                    h_v[i, kk, pl.ds(f * 16, 16)] = accs[f]
            plsc.parallel_loop(0, K, unroll=2)(k_body)
        pltpu.sync_copy(h_v, h_hbm.at[pl.ds((base + t) * C, C)])

    prefetch(0, 0)
    prefetch(1, 1)

    def m_body(m, carry):
        t0 = 2 * m
        wait_rows(0)
        compute(t0, 0)
        prefetch(t0 + 2, 0)
        wait_rows(1)
        compute(t0 + 1, 1)
        prefetch(t0 + 3, 1)
        return carry

    lax.fori_loop(0, CHUNKS_PER_W // 2, m_body, None)
    wait_rows(0)
    wait_rows(1)


def _sc_gather(x, gidx, wflat):
    return pl.kernel(
        _sc_body,
        out_type=jax.ShapeDtypeStruct((N_PAD, K, FEATS), jnp.float32),
        mesh=plsc.VectorSubcoreMesh(core_axis_name="c", subcore_axis_name="s"),
        scratch_types=[
            pltpu.VMEM((C, E_PAD), jnp.int32),
            pltpu.VMEM((C, E_PAD), jnp.int32),
            pltpu.VMEM((C, W_PAD), jnp.float32),
            pltpu.VMEM((C, W_PAD), jnp.float32),
            pltpu.VMEM((C, E_PAD, FEATS), jnp.float32),
            pltpu.VMEM((C, E_PAD, FEATS), jnp.float32),
            pltpu.VMEM((C, K, FEATS), jnp.float32),
            pltpu.SemaphoreType.DMA,
            pltpu.SemaphoreType.DMA,
        ],
    )(x, gidx, wflat)


def kernel(x, neigh_indices, neigh_weights, W, b):
    n = x.shape[0]
    gidx = neigh_indices - 1
    gidx = jnp.pad(gidx, ((0, N_PAD - n), (0, E_PAD - E_RAW)))
    wflat = neigh_weights.reshape(n, E_RAW)
    wflat = jnp.pad(wflat, ((0, N_PAD - n), (0, W_PAD - E_RAW)))

    h = _sc_gather(x,
                   gidx.reshape(CHUNKS, C, E_PAD),
                   wflat.reshape(CHUNKS, C, W_PAD))

    out = _tc_matmul(h.reshape(N_PAD, K * FEATS), W.T, b.reshape(1, FEATS))
    return out[:n]


# speedup vs baseline: 5.9892x; 5.5478x over previous
import jax
import jax.numpy as jnp
from jax import lax
from jax.experimental import pallas as pl
from jax.experimental.pallas import tpu as pltpu
from jax.experimental.pallas import tpu_sc as plsc

N_NODES = 10000
FEATS = 128
K = 25
E_RAW = 75
E_PAD = 80
W_PAD = 96

NW = 32
C = 2
N_PAD = 10240
CHUNKS = N_PAD // C
CHUNKS_PER_W = CHUNKS // NW


def _mm_body(h_ref, wt_ref, b_ref, o_ref):
    o_ref[...] = jnp.dot(h_ref[...], wt_ref[...],
                         preferred_element_type=jnp.float32) + b_ref[...]


def _tc_matmul(h, wt, b2):
    blk = 1024
    nb = h.shape[0] // blk
    kk = h.shape[1]
    return pl.pallas_call(
        _mm_body,
        grid=(nb,),
        in_specs=[
            pl.BlockSpec((blk, kk), lambda i: (i, 0)),
            pl.BlockSpec((kk, FEATS), lambda i: (0, 0)),
            pl.BlockSpec((1, FEATS), lambda i: (0, 0)),
        ],
        out_specs=pl.BlockSpec((blk, FEATS), lambda i: (i, 0)),
        out_shape=jax.ShapeDtypeStruct((h.shape[0], FEATS), jnp.float32),
    )(h, wt, b2)


def _sc_body(x_hbm, gidx_hbm, w_hbm, h_hbm,
             idx0, idx1, w0, w1, rows0, rows1, h_v, x_sh, sem0, sem1):
    sid = lax.axis_index("s")
    wid = sid * 2 + lax.axis_index("c")
    base = wid * CHUNKS_PER_W
    idx_v = (idx0, idx1)
    w_v = (w0, w1)
    rows_v = (rows0, rows1)
    sem = (sem0, sem1)

    @pl.when(sid == 0)
    def _stage():
        pltpu.sync_copy(x_hbm, x_sh)
    plsc.subcore_barrier()

    lane = jnp.arange(16, dtype=jnp.int32)
    rotidx = [((lane + p) % 3)[:, None] for p in range(3)]
    gdn = lax.GatherDimensionNumbers(
        offset_dims=(), collapsed_slice_dims=(0,), start_index_map=(0,))

    def lane_gather(vec, idx):
        return lax.gather(vec, idx, dimension_numbers=gdn, slice_sizes=(1,),
                          mode=lax.GatherScatterMode.PROMISE_IN_BOUNDS)

    def prefetch(t, buf):
        chunk = jnp.minimum(base + t, CHUNKS - 1)
        pltpu.sync_copy(gidx_hbm.at[chunk], idx_v[buf])
        pltpu.sync_copy(w_hbm.at[chunk], w_v[buf])
        for j in range(C):
            pltpu.async_copy(x_sh.at[idx_v[buf].at[j]],
                             rows_v[buf].at[j], sem[buf])

    def wait_rows(buf):
        for j in range(C):
            pltpu.make_async_copy(x_sh.at[idx_v[buf].at[j]],
                                  rows_v[buf].at[j], sem[buf]).wait()

    def compute(t, buf):
        for i in range(C):
            def k_body(kk):
                wv = w_v[buf][i, pl.ds(kk * 3, 16)]
                V = [lane_gather(wv, rotidx[p]) for p in range(3)]
                accs = []
                for f in range(8):
                    r0 = rows_v[buf][i, kk * 3, pl.ds(f * 16, 16)]
                    r1 = rows_v[buf][i, kk * 3 + 1, pl.ds(f * 16, 16)]
                    r2 = rows_v[buf][i, kk * 3 + 2, pl.ds(f * 16, 16)]
                    accs.append(V[f % 3] * r0
                                + V[(f + 2) % 3] * r1
                                + V[(f + 1) % 3] * r2)
                for f in range(8):
                    h_v[i, kk, pl.ds(f * 16, 16)] = accs[f]
            plsc.parallel_loop(0, K, unroll=2)(k_body)
        pltpu.sync_copy(h_v, h_hbm.at[pl.ds((base + t) * C, C)])

    prefetch(0, 0)
    prefetch(1, 1)

    def m_body(m, carry):
        t0 = 2 * m
        wait_rows(0)
        compute(t0, 0)
        prefetch(t0 + 2, 0)
        wait_rows(1)
        compute(t0 + 1, 1)
        prefetch(t0 + 3, 1)
        return carry

    lax.fori_loop(0, CHUNKS_PER_W // 2, m_body, None)
    wait_rows(0)
    wait_rows(1)


def _sc_gather(x, gidx, wflat):
    return pl.kernel(
        _sc_body,
        out_type=jax.ShapeDtypeStruct((N_PAD, K, FEATS), jnp.float32),
        mesh=plsc.VectorSubcoreMesh(core_axis_name="c", subcore_axis_name="s"),
        scratch_types=[
            pltpu.VMEM((C, E_PAD), jnp.int32),
            pltpu.VMEM((C, E_PAD), jnp.int32),
            pltpu.VMEM((C, W_PAD), jnp.float32),
            pltpu.VMEM((C, W_PAD), jnp.float32),
            pltpu.VMEM((C, E_PAD, FEATS), jnp.float32),
            pltpu.VMEM((C, E_PAD, FEATS), jnp.float32),
            pltpu.VMEM((C, K, FEATS), jnp.float32),
            pltpu.VMEM_SHARED((N_NODES, FEATS), jnp.float32),
            pltpu.SemaphoreType.DMA,
            pltpu.SemaphoreType.DMA,
        ],
    )(x, gidx, wflat)


def kernel(x, neigh_indices, neigh_weights, W, b):
    n = x.shape[0]
    gidx = neigh_indices - 1
    gidx = jnp.pad(gidx, ((0, N_PAD - n), (0, E_PAD - E_RAW)))
    wflat = neigh_weights.reshape(n, E_RAW)
    wflat = jnp.pad(wflat, ((0, N_PAD - n), (0, W_PAD - E_RAW)))

    h = _sc_gather(x,
                   gidx.reshape(CHUNKS, C, E_PAD),
                   wflat.reshape(CHUNKS, C, W_PAD))

    out = _tc_matmul(h.reshape(N_PAD, K * FEATS), W.T, b.reshape(1, FEATS))
    return out[:n]


# speedup vs baseline: 9.5493x; 1.5944x over previous
import jax
import jax.numpy as jnp
from jax import lax
from jax.experimental import pallas as pl
from jax.experimental.pallas import tpu as pltpu
from jax.experimental.pallas import tpu_sc as plsc

N_NODES = 10000
FEATS = 128
K = 25
E_RAW = 75
E_PAD = 80
W_PAD = 96

NW = 32
C = 2
N_PAD = 10240
CHUNKS = N_PAD // C
CHUNKS_PER_W = CHUNKS // NW


def _mm_body(h_ref, wt_ref, b_ref, o_ref):
    o_ref[...] = jnp.dot(h_ref[...], wt_ref[...],
                         preferred_element_type=jnp.float32) + b_ref[...]


def _tc_matmul(h, wt, b2):
    blk = 1024
    nb = h.shape[0] // blk
    kk = h.shape[1]
    return pl.pallas_call(
        _mm_body,
        grid=(nb,),
        in_specs=[
            pl.BlockSpec((blk, kk), lambda i: (i, 0)),
            pl.BlockSpec((kk, FEATS), lambda i: (0, 0)),
            pl.BlockSpec((1, FEATS), lambda i: (0, 0)),
        ],
        out_specs=pl.BlockSpec((blk, FEATS), lambda i: (i, 0)),
        out_shape=jax.ShapeDtypeStruct((h.shape[0], FEATS), jnp.float32),
    )(h, wt, b2)


def _sc_body(x_hbm, gidx_hbm, w_hbm, h_hbm,
             idx0, idx1, w0, w1, rows0, rows1, h_v, x_sh, sem0, sem1):
    sid = lax.axis_index("s")
    wid = sid * 2 + lax.axis_index("c")
    base = wid * CHUNKS_PER_W
    idx_v = (idx0, idx1)
    w_v = (w0, w1)
    rows_v = (rows0, rows1)
    sem = (sem0, sem1)

    @pl.when(sid == 0)
    def _stage():
        pltpu.sync_copy(x_hbm, x_sh)
    plsc.subcore_barrier()

    lane = jnp.arange(16, dtype=jnp.int32)
    rotidx = [((lane + p) % 3)[:, None] for p in range(3)]
    gdn = lax.GatherDimensionNumbers(
        offset_dims=(), collapsed_slice_dims=(0,), start_index_map=(0,))

    def lane_gather(vec, idx):
        return lax.gather(vec, idx, dimension_numbers=gdn, slice_sizes=(1,),
                          mode=lax.GatherScatterMode.PROMISE_IN_BOUNDS)

    def prefetch(t, buf):
        chunk = jnp.minimum(base + t, CHUNKS - 1)
        pltpu.sync_copy(gidx_hbm.at[chunk], idx_v[buf])
        pltpu.sync_copy(w_hbm.at[chunk], w_v[buf])
        for j in range(C):
            pltpu.async_copy(x_sh.at[idx_v[buf].at[j]],
                             rows_v[buf].at[j], sem[buf])

    def wait_rows(buf):
        for j in range(C):
            pltpu.make_async_copy(x_sh.at[idx_v[buf].at[j]],
                                  rows_v[buf].at[j], sem[buf]).wait()

    def compute(t, buf):
        for i in range(C):
            def k_body(kk):
                wv = w_v[buf][i, pl.ds(kk * 3, 16)]
                V = [lane_gather(wv, rotidx[p]) for p in range(3)]
                accs = []
                for f in range(8):
                    r0 = rows_v[buf][i, kk * 3, pl.ds(f * 16, 16)]
                    r1 = rows_v[buf][i, kk * 3 + 1, pl.ds(f * 16, 16)]
                    r2 = rows_v[buf][i, kk * 3 + 2, pl.ds(f * 16, 16)]
                    accs.append(V[f % 3] * r0
                                + V[(f + 2) % 3] * r1
                                + V[(f + 1) % 3] * r2)
                for f in range(8):
                    h_v[i, pl.ds(kk * FEATS + f * 16, 16)] = accs[f]
            plsc.parallel_loop(0, K, unroll=2)(k_body)
        pltpu.sync_copy(h_v, h_hbm.at[pl.ds((base + t) * C, C)])

    prefetch(0, 0)
    prefetch(1, 1)

    def m_body(m, carry):
        t0 = 2 * m
        wait_rows(0)
        compute(t0, 0)
        prefetch(t0 + 2, 0)
        wait_rows(1)
        compute(t0 + 1, 1)
        prefetch(t0 + 3, 1)
        return carry

    lax.fori_loop(0, CHUNKS_PER_W // 2, m_body, None)
    wait_rows(0)
    wait_rows(1)


def _sc_gather(x, gidx, wflat):
    return pl.kernel(
        _sc_body,
        out_type=jax.ShapeDtypeStruct((N_PAD, K * FEATS), jnp.float32),
        mesh=plsc.VectorSubcoreMesh(core_axis_name="c", subcore_axis_name="s"),
        scratch_types=[
            pltpu.VMEM((C, E_PAD), jnp.int32),
            pltpu.VMEM((C, E_PAD), jnp.int32),
            pltpu.VMEM((C, W_PAD), jnp.float32),
            pltpu.VMEM((C, W_PAD), jnp.float32),
            pltpu.VMEM((C, E_PAD, FEATS), jnp.float32),
            pltpu.VMEM((C, E_PAD, FEATS), jnp.float32),
            pltpu.VMEM((C, K * FEATS), jnp.float32),
            pltpu.VMEM_SHARED((N_NODES, FEATS), jnp.float32),
            pltpu.SemaphoreType.DMA,
            pltpu.SemaphoreType.DMA,
        ],
    )(x, gidx, wflat)


def kernel(x, neigh_indices, neigh_weights, W, b):
    n = x.shape[0]
    gidx = neigh_indices - 1
    gidx = jnp.pad(gidx, ((0, N_PAD - n), (0, E_PAD - E_RAW)))
    wflat = neigh_weights.reshape(n, E_RAW)
    wflat = jnp.pad(wflat, ((0, N_PAD - n), (0, W_PAD - E_RAW)))

    h = _sc_gather(x,
                   gidx.reshape(CHUNKS, C, E_PAD),
                   wflat.reshape(CHUNKS, C, W_PAD))

    out = _tc_matmul(h, W.T, b.reshape(1, FEATS))
    return out[:n]


# speedup vs baseline: 11.4865x; 1.2029x over previous
import jax
import jax.numpy as jnp
from jax import lax
from jax.experimental import pallas as pl
from jax.experimental.pallas import tpu as pltpu
from jax.experimental.pallas import tpu_sc as plsc

N_NODES = 10000
FEATS = 128
K = 25
E_RAW = 75
E_PAD = 80
W_PAD = 96

NW = 32
C = 2
N_PAD = 10240
CHUNKS = N_PAD // C
CHUNKS_PER_W = CHUNKS // NW


def _mm_body(h_ref, wt_ref, b_ref, o_ref):
    o_ref[...] = jnp.dot(h_ref[...], wt_ref[...],
                         preferred_element_type=jnp.float32) + b_ref[...]


def _tc_matmul(h, wt, b2):
    blk = 1024
    nb = h.shape[0] // blk
    kk = h.shape[1]
    return pl.pallas_call(
        _mm_body,
        grid=(nb,),
        in_specs=[
            pl.BlockSpec((blk, kk), lambda i: (i, 0)),
            pl.BlockSpec((kk, FEATS), lambda i: (0, 0)),
            pl.BlockSpec((1, FEATS), lambda i: (0, 0)),
        ],
        out_specs=pl.BlockSpec((blk, FEATS), lambda i: (i, 0)),
        out_shape=jax.ShapeDtypeStruct((h.shape[0], FEATS), jnp.float32),
    )(h, wt, b2)


def _sc_body(x_hbm, gidx_hbm, w_hbm, h_hbm,
             idx0, idx1, w0, w1, rows0, rows1, h_v, x_sh,
             sem0, sem1, wsem0, wsem1):
    sid = lax.axis_index("s")
    wid = sid * 2 + lax.axis_index("c")
    base = wid * CHUNKS_PER_W
    idx_v = (idx0, idx1)
    w_v = (w0, w1)
    rows_v = (rows0, rows1)
    sem = (sem0, sem1)
    wsem = (wsem0, wsem1)

    @pl.when(sid == 0)
    def _stage():
        pltpu.sync_copy(x_hbm, x_sh)
    plsc.subcore_barrier()

    lane = jnp.arange(16, dtype=jnp.int32)
    rotidx = [((lane + p) % 3)[:, None] for p in range(3)]
    gdn = lax.GatherDimensionNumbers(
        offset_dims=(), collapsed_slice_dims=(0,), start_index_map=(0,))

    def lane_gather(vec, idx):
        return lax.gather(vec, idx, dimension_numbers=gdn, slice_sizes=(1,),
                          mode=lax.GatherScatterMode.PROMISE_IN_BOUNDS)

    def prefetch(t, buf):
        chunk = jnp.minimum(base + t, CHUNKS - 1)
        pltpu.sync_copy(gidx_hbm.at[chunk], idx_v[buf])
        pltpu.async_copy(w_hbm.at[chunk], w_v[buf], wsem[buf])
        for j in range(C):
            pltpu.async_copy(x_sh.at[idx_v[buf].at[j]],
                             rows_v[buf].at[j], sem[buf])

    def wait_rows(buf):
        pltpu.make_async_copy(w_hbm.at[0], w_v[buf], wsem[buf]).wait()
        for j in range(C):
            pltpu.make_async_copy(x_sh.at[idx_v[buf].at[j]],
                                  rows_v[buf].at[j], sem[buf]).wait()

    def compute(t, buf):
        def k_body(kk):
            for i in range(C):
                wv = w_v[buf][i, pl.ds(kk * 3, 16)]
                V = [lane_gather(wv, rotidx[p]) for p in range(3)]
                accs = []
                for f in range(8):
                    r0 = rows_v[buf][i, kk * 3, pl.ds(f * 16, 16)]
                    r1 = rows_v[buf][i, kk * 3 + 1, pl.ds(f * 16, 16)]
                    r2 = rows_v[buf][i, kk * 3 + 2, pl.ds(f * 16, 16)]
                    accs.append(V[f % 3] * r0
                                + V[(f + 2) % 3] * r1
                                + V[(f + 1) % 3] * r2)
                for f in range(8):
                    h_v[i, pl.ds(kk * FEATS + f * 16, 16)] = accs[f]
        plsc.parallel_loop(0, K, unroll=2)(k_body)
        pltpu.sync_copy(h_v, h_hbm.at[pl.ds((base + t) * C, C)])

    prefetch(0, 0)
    prefetch(1, 1)

    def m_body(m, carry):
        t0 = 2 * m
        wait_rows(0)
        compute(t0, 0)
        prefetch(t0 + 2, 0)
        wait_rows(1)
        compute(t0 + 1, 1)
        prefetch(t0 + 3, 1)
        return carry

    lax.fori_loop(0, CHUNKS_PER_W // 2, m_body, None)
    wait_rows(0)
    wait_rows(1)


def _sc_gather(x, gidx, wflat):
    return pl.kernel(
        _sc_body,
        out_type=jax.ShapeDtypeStruct((N_PAD, K * FEATS), jnp.float32),
        mesh=plsc.VectorSubcoreMesh(core_axis_name="c", subcore_axis_name="s"),
        scratch_types=[
            pltpu.VMEM((C, E_PAD), jnp.int32),
            pltpu.VMEM((C, E_PAD), jnp.int32),
            pltpu.VMEM((C, W_PAD), jnp.float32),
            pltpu.VMEM((C, W_PAD), jnp.float32),
            pltpu.VMEM((C, E_PAD, FEATS), jnp.float32),
            pltpu.VMEM((C, E_PAD, FEATS), jnp.float32),
            pltpu.VMEM((C, K * FEATS), jnp.float32),
            pltpu.VMEM_SHARED((N_NODES, FEATS), jnp.float32),
            pltpu.SemaphoreType.DMA,
            pltpu.SemaphoreType.DMA,
            pltpu.SemaphoreType.DMA,
            pltpu.SemaphoreType.DMA,
        ],
    )(x, gidx, wflat)


def kernel(x, neigh_indices, neigh_weights, W, b):
    n = x.shape[0]
    gidx = neigh_indices - 1
    gidx = jnp.pad(gidx, ((0, N_PAD - n), (0, E_PAD - E_RAW)))
    wflat = neigh_weights.reshape(n, E_RAW)
    wflat = jnp.pad(wflat, ((0, N_PAD - n), (0, W_PAD - E_RAW)))

    h = _sc_gather(x,
                   gidx.reshape(CHUNKS, C, E_PAD),
                   wflat.reshape(CHUNKS, C, W_PAD))

    out = _tc_matmul(h, W.T, b.reshape(1, FEATS))
    return out[:n]


# speedup vs baseline: 13.3169x; 1.1594x over previous
import jax
import jax.numpy as jnp
from jax import lax
from jax.experimental import pallas as pl
from jax.experimental.pallas import tpu as pltpu
from jax.experimental.pallas import tpu_sc as plsc

N_NODES = 10000
FEATS = 128
K = 25
E_RAW = 75
E_PAD = 80
W_PAD = 96

NW = 32
C = 2
N_PAD = 10240
CHUNKS = N_PAD // C
CHUNKS_PER_W = CHUNKS // NW


def _mm_body(h_ref, wt_ref, b_ref, o_ref):
    o_ref[...] = jnp.dot(h_ref[...], wt_ref[...],
                         preferred_element_type=jnp.float32) + b_ref[...]


def _tc_matmul(h, wt, b2, n):
    blk = 1000
    nb = n // blk
    kk = h.shape[1]
    return pl.pallas_call(
        _mm_body,
        grid=(nb,),
        in_specs=[
            pl.BlockSpec((blk, kk), lambda i: (i, 0)),
            pl.BlockSpec((kk, FEATS), lambda i: (0, 0)),
            pl.BlockSpec((1, FEATS), lambda i: (0, 0)),
        ],
        out_specs=pl.BlockSpec((blk, FEATS), lambda i: (i, 0)),
        out_shape=jax.ShapeDtypeStruct((n, FEATS), jnp.float32),
    )(h, wt, b2)


def _sc_body(x_hbm, gidx_hbm, w_hbm, h_hbm,
             idxA, idxB, wA, wB, rows0, rows1, h_v, x_sh,
             sem0, sem1, wsemA, wsemB):
    sid = lax.axis_index("s")
    wid = sid * 2 + lax.axis_index("c")
    base = wid * CHUNKS_PER_W
    idx_v = (idxA, idxB)
    w_v = (wA, wB)
    rows_v = (rows0, rows1)
    sem = (sem0, sem1)
    wsem = (wsemA, wsemB)

    @pl.when(sid == 0)
    def _stage():
        pltpu.sync_copy(x_hbm, x_sh)
    plsc.subcore_barrier()

    lane = jnp.arange(16, dtype=jnp.int32)
    rotidx = [((lane + p) % 3)[:, None] for p in range(3)]
    gdn = lax.GatherDimensionNumbers(
        offset_dims=(), collapsed_slice_dims=(0,), start_index_map=(0,))

    def lane_gather(vec, idx):
        return lax.gather(vec, idx, dimension_numbers=gdn, slice_sizes=(1,),
                          mode=lax.GatherScatterMode.PROMISE_IN_BOUNDS)

    def fetch_pair(half, ab):
        pair = jnp.minimum(base // 2 + half, CHUNKS // 2 - 1)
        pltpu.sync_copy(gidx_hbm.at[pair], idx_v[ab])
        pltpu.async_copy(w_hbm.at[pair], w_v[ab], wsem[ab])

    def fire_rows(ab, p, buf):
        for j in range(C):
            pltpu.async_copy(x_sh.at[idx_v[ab].at[p, j, pl.ds(0, E_RAW)]],
                             rows_v[buf].at[j], sem[buf])

    def wait_rows(ab, p, buf):
        for j in range(C):
            pltpu.make_async_copy(
                x_sh.at[idx_v[ab].at[p, j, pl.ds(0, E_RAW)]],
                rows_v[buf].at[j], sem[buf]).wait()

    def wait_w(ab):
        pltpu.make_async_copy(w_hbm.at[0], w_v[ab], wsem[ab]).wait()

    def compute(t, ab, p, buf):
        def k_body(kk):
            for i in range(C):
                wv = w_v[ab][p, i, pl.ds(kk * 3, 16)]
                V = [lane_gather(wv, rotidx[q]) for q in range(3)]
                accs = []
                for f in range(8):
                    r0 = rows_v[buf][i, kk * 3, pl.ds(f * 16, 16)]
                    r1 = rows_v[buf][i, kk * 3 + 1, pl.ds(f * 16, 16)]
                    r2 = rows_v[buf][i, kk * 3 + 2, pl.ds(f * 16, 16)]
                    accs.append(V[f % 3] * r0
                                + V[(f + 2) % 3] * r1
                                + V[(f + 1) % 3] * r2)
                for f in range(8):
                    h_v[i, pl.ds(kk * FEATS + f * 16, 16)] = accs[f]
        plsc.parallel_loop(0, K, unroll=2)(k_body)
        pltpu.sync_copy(h_v, h_hbm.at[pl.ds((base + t) * C, C)])

    fetch_pair(0, 0)
    fire_rows(0, 0, 0)
    fire_rows(0, 1, 1)

    def m_body(m, carry):
        t0 = 4 * m
        fetch_pair(2 * m + 1, 1)
        wait_w(0)
        wait_rows(0, 0, 0)
        compute(t0, 0, 0, 0)
        fire_rows(1, 0, 0)
        wait_rows(0, 1, 1)
        compute(t0 + 1, 0, 1, 1)
        fire_rows(1, 1, 1)
        fetch_pair(2 * m + 2, 0)
        wait_w(1)
        wait_rows(1, 0, 0)
        compute(t0 + 2, 1, 0, 0)
        fire_rows(0, 0, 0)
        wait_rows(1, 1, 1)
        compute(t0 + 3, 1, 1, 1)
        fire_rows(0, 1, 1)
        return carry

    lax.fori_loop(0, CHUNKS_PER_W // 4, m_body, None)
    wait_w(0)
    wait_rows(0, 0, 0)
    wait_rows(0, 1, 1)


def _sc_gather(x, gidx, wflat):
    return pl.kernel(
        _sc_body,
        out_type=jax.ShapeDtypeStruct((N_PAD, K * FEATS), jnp.float32),
        mesh=plsc.VectorSubcoreMesh(core_axis_name="c", subcore_axis_name="s"),
        scratch_types=[
            pltpu.VMEM((2, C, E_PAD), jnp.int32),
            pltpu.VMEM((2, C, E_PAD), jnp.int32),
            pltpu.VMEM((2, C, W_PAD), jnp.float32),
            pltpu.VMEM((2, C, W_PAD), jnp.float32),
            pltpu.VMEM((C, E_RAW, FEATS), jnp.float32),
            pltpu.VMEM((C, E_RAW, FEATS), jnp.float32),
            pltpu.VMEM((C, K * FEATS), jnp.float32),
            pltpu.VMEM_SHARED((N_NODES, FEATS), jnp.float32),
            pltpu.SemaphoreType.DMA,
            pltpu.SemaphoreType.DMA,
            pltpu.SemaphoreType.DMA,
            pltpu.SemaphoreType.DMA,
        ],
    )(x, gidx, wflat)


def kernel(x, neigh_indices, neigh_weights, W, b):
    n = x.shape[0]
    gidx = jnp.pad(neigh_indices - 1, ((0, N_PAD - n), (0, E_PAD - E_RAW)))
    wflat = jnp.pad(neigh_weights.reshape(n, E_RAW),
                    ((0, N_PAD - n), (0, W_PAD - E_RAW)))

    h = _sc_gather(x,
                   gidx.reshape(CHUNKS // 2, 2, C, E_PAD),
                   wflat.reshape(CHUNKS // 2, 2, C, W_PAD))

    return _tc_matmul(h, W.T, b.reshape(1, FEATS), n)
